# Initial kernel scaffold; baseline (speedup 1.0000x reference)
import jax
import jax.numpy as jnp
from jax.experimental import pallas as pl


def kernel(x, weight, bias):
    raise NotImplementedError("write your pallas kernel here")


# speedup vs baseline: 4.1136x; 4.1136x over previous
import jax
import jax.numpy as jnp
from jax.experimental import pallas as pl
from jax.experimental.pallas import tpu as pltpu

C_IN = 32
C_OUT = 32
MIN_V = 0.1
MAX_V = 0.8


def _conv_taps_kernel(xo_ref, w_ref, b_ref, o_ref, *, wo):
    xo = xo_ref[0]
    m = xo.shape[1]
    lane = jax.lax.broadcasted_iota(jnp.int32, (C_IN, m), 1)
    interior = (lane % wo) != 0
    zc = jnp.zeros((C_IN, wo + 1), xo.dtype)
    t01 = jnp.concatenate([zc[:, :wo], xo[:, :-wo]], axis=1)
    t10 = jnp.where(interior,
                    jnp.concatenate([zc[:, :1], xo[:, :-1]], axis=1), 0.0)
    t00 = jnp.where(interior,
                    jnp.concatenate([zc, xo[:, :-(wo + 1)]], axis=1), 0.0)
    acc = jnp.dot(w_ref[0], t00, preferred_element_type=jnp.float32)
    acc = acc + jnp.dot(w_ref[1], t01, preferred_element_type=jnp.float32)
    acc = acc + jnp.dot(w_ref[2], t10, preferred_element_type=jnp.float32)
    acc = acc + jnp.dot(w_ref[3], xo, preferred_element_type=jnp.float32)
    o_ref[0] = jnp.clip(acc + b_ref[...], MIN_V, MAX_V)


def kernel(x, weight, bias):
    n, _, h, w = x.shape
    ho, wo = h // 2, w // 2
    m = ho * wo

    xo = x[:, :, 1::2, 1::2].reshape(n, C_IN, m)
    wt = jnp.transpose(weight, (2, 3, 0, 1)).reshape(4, C_OUT, C_IN)
    b2 = bias.reshape(C_OUT, 1).astype(jnp.float32)

    import functools
    out = pl.pallas_call(
        functools.partial(_conv_taps_kernel, wo=wo),
        out_shape=jax.ShapeDtypeStruct((n, C_OUT, m), jnp.float32),
        grid=(n,),
        in_specs=[
            pl.BlockSpec((1, C_IN, m), lambda i: (i, 0, 0)),
            pl.BlockSpec((4, C_OUT, C_IN), lambda i: (0, 0, 0)),
            pl.BlockSpec((C_OUT, 1), lambda i: (0, 0)),
        ],
        out_specs=pl.BlockSpec((1, C_OUT, m), lambda i: (i, 0, 0)),
        compiler_params=pltpu.CompilerParams(
            dimension_semantics=("parallel",)),
    )(xo, wt, b2)
    return out.reshape(n, C_OUT, ho, wo)


# speedup vs baseline: 17.9794x; 4.3707x over previous
import functools

import jax
import jax.numpy as jnp
from jax.experimental import pallas as pl
from jax.experimental.pallas import tpu as pltpu

C_IN = 32
C_OUT = 32
MIN_V = 0.1
MAX_V = 0.8


def _conv_taps_kernel(x_ref, s_ref, w_ref, b_ref, o_ref, *, ho, wo):
    w = 2 * wo
    a2 = x_ref[0].reshape(C_IN * ho, w)
    p = jnp.dot(a2, s_ref[...], preferred_element_type=jnp.float32)
    p3 = p.reshape(C_IN, ho, 2 * wo)
    tl = p3[:, :, :wo].reshape(C_IN, ho * wo)
    tr = p3[:, :, wo:].reshape(C_IN, ho * wo)
    zc = jnp.zeros((C_IN, wo), jnp.float32)
    tlu = jnp.concatenate([zc, tl[:, :-wo]], axis=1)
    tru = jnp.concatenate([zc, tr[:, :-wo]], axis=1)
    acc = jnp.dot(w_ref[0], tlu, preferred_element_type=jnp.float32)
    acc = acc + jnp.dot(w_ref[1], tru, preferred_element_type=jnp.float32)
    acc = acc + jnp.dot(w_ref[2], tl, preferred_element_type=jnp.float32)
    acc = acc + jnp.dot(w_ref[3], tr, preferred_element_type=jnp.float32)
    o_ref[0] = jnp.clip(acc + b_ref[...], MIN_V, MAX_V)


def kernel(x, weight, bias):
    n, _, h, w = x.shape
    ho, wo = h // 2, w // 2
    m = ho * wo

    xv = x.reshape(n, C_IN, ho, 2 * w)
    wt = jnp.transpose(weight, (2, 3, 0, 1)).reshape(4, C_OUT, C_IN)
    b2 = bias.reshape(C_OUT, 1).astype(jnp.float32)
    rows = jnp.arange(w)[:, None]
    cols = jnp.arange(2 * wo)[None, :]
    sel = (rows == (2 * (cols % wo) - 1 + 2 * (cols // wo))).astype(jnp.float32)

    out = pl.pallas_call(
        functools.partial(_conv_taps_kernel, ho=ho, wo=wo),
        out_shape=jax.ShapeDtypeStruct((n, C_OUT, m), jnp.float32),
        grid=(n,),
        in_specs=[
            pl.BlockSpec((1, C_IN, ho, w), lambda i: (i, 0, 0, 1)),
            pl.BlockSpec((w, 2 * wo), lambda i: (0, 0)),
            pl.BlockSpec((4, C_OUT, C_IN), lambda i: (0, 0, 0)),
            pl.BlockSpec((C_OUT, 1), lambda i: (0, 0)),
        ],
        out_specs=pl.BlockSpec((1, C_OUT, m), lambda i: (i, 0, 0)),
        compiler_params=pltpu.CompilerParams(
            dimension_semantics=("parallel",)),
    )(xv, sel, wt, b2)
    return out.reshape(n, C_OUT, ho, wo)


# speedup vs baseline: 44.3740x; 2.4680x over previous
import functools

import jax
import jax.numpy as jnp
from jax.experimental import pallas as pl
from jax.experimental.pallas import tpu as pltpu

C_IN = 32
C_OUT = 32
MIN_V = 0.1
MAX_V = 0.8


def _conv_taps_kernel(x_ref, s_ref, w_ref, b_ref, o_ref, *, ho, wo):
    xr = x_ref[0, :, 1 : 2 * ho : 2, :]
    a2 = xr.reshape(C_IN * ho, 2 * wo)
    p = jnp.dot(a2, s_ref[...], preferred_element_type=jnp.float32)
    p3 = p.reshape(C_IN, ho, 2 * wo)
    tl = p3[:, :, :wo].reshape(C_IN, ho * wo)
    tr = p3[:, :, wo:].reshape(C_IN, ho * wo)
    zc = jnp.zeros((C_IN, wo), jnp.float32)
    tlu = jnp.concatenate([zc, tl[:, :-wo]], axis=1)
    tru = jnp.concatenate([zc, tr[:, :-wo]], axis=1)
    acc = jnp.dot(w_ref[0], tlu, preferred_element_type=jnp.float32)
    acc = acc + jnp.dot(w_ref[1], tru, preferred_element_type=jnp.float32)
    acc = acc + jnp.dot(w_ref[2], tl, preferred_element_type=jnp.float32)
    acc = acc + jnp.dot(w_ref[3], tr, preferred_element_type=jnp.float32)
    acc = jnp.clip(acc + b_ref[...], MIN_V, MAX_V)
    o_ref[0] = acc.reshape(C_OUT, ho, wo)


def kernel(x, weight, bias):
    n, _, h, w = x.shape
    ho, wo = h // 2, w // 2

    wt = jnp.transpose(weight, (2, 3, 0, 1)).reshape(4, C_OUT, C_IN)
    b2 = bias.reshape(C_OUT, 1).astype(jnp.float32)
    rows = jnp.arange(w)[:, None]
    cols = jnp.arange(2 * wo)[None, :]
    sel = (rows == (2 * (cols % wo) - 1 + 2 * (cols // wo))).astype(jnp.float32)

    return pl.pallas_call(
        functools.partial(_conv_taps_kernel, ho=ho, wo=wo),
        out_shape=jax.ShapeDtypeStruct((n, C_OUT, ho, wo), jnp.float32),
        grid=(n,),
        in_specs=[
            pl.BlockSpec((1, C_IN, h, w), lambda i: (i, 0, 0, 0)),
            pl.BlockSpec((w, 2 * wo), lambda i: (0, 0)),
            pl.BlockSpec((4, C_OUT, C_IN), lambda i: (0, 0, 0)),
            pl.BlockSpec((C_OUT, 1), lambda i: (0, 0)),
        ],
        out_specs=pl.BlockSpec((1, C_OUT, ho, wo), lambda i: (i, 0, 0, 0)),
        compiler_params=pltpu.CompilerParams(
            dimension_semantics=("parallel",)),
    )(x, sel, wt, b2)
